# baseline (speedup 1.0000x reference)
import functools

import jax
import jax.numpy as jnp
from jax import lax
from jax.experimental import pallas as pl
from jax.experimental.pallas import tpu as pltpu
from jax.experimental.pallas import tpu_sc as plsc

L = 16
NSUB = 16
NCORE = 2
CH = 88
NBUF = 4


def _sc_mesh():
    return plsc.VectorSubcoreMesh(
        core_axis_name="c", subcore_axis_name="s",
        num_cores=NCORE, num_subcores=NSUB,
    )


CL = 128


def _make_degree_kernel(Epad, NP):
    cpt = Epad // (NSUB * CH)

    @functools.partial(
        pl.kernel,
        out_type=jax.ShapeDtypeStruct((2 * NP, CL), jnp.float32),
        mesh=_sc_mesh(),
        scratch_types=[
            pltpu.VMEM((CH,), jnp.int32),
            pltpu.VMEM((CH, CL), jnp.float32),
            pltpu.VMEM((CH, CL), jnp.float32),
            pltpu.VMEM_SHARED((NP, CL), jnp.float32),
        ],
    )
    def deg_kernel(edges_hbm, out_hbm, idx_v, ones_v, zero_v, bins_sh):
        cid = lax.axis_index("c")
        sid = lax.axis_index("s")
        rows_per_tile = NP // NSUB

        def fill(i, _):
            for j in range(CL // L):
                ones_v[i, pl.ds(j * L, L)] = jnp.full((L,), 1.0, jnp.float32)
                zero_v[i, pl.ds(j * L, L)] = jnp.zeros((L,), jnp.float32)
            return 0

        lax.fori_loop(0, CH, fill, 0)
        for k in range(0, rows_per_tile, CH):
            n = min(CH, rows_per_tile - k)
            pltpu.sync_copy(zero_v.at[pl.ds(0, n)],
                            bins_sh.at[pl.ds(sid * rows_per_tile + k, n)])
        plsc.subcore_barrier()

        ebase = cid * Epad + sid * (cpt * CH)

        def body(i, _):
            pltpu.sync_copy(edges_hbm.at[pl.ds(ebase + i * CH, CH)], idx_v)
            pltpu.sync_copy(ones_v, bins_sh.at[idx_v], add=True)
            return 0

        lax.fori_loop(0, cpt, body, 0)
        plsc.subcore_barrier()
        pltpu.sync_copy(
            bins_sh.at[pl.ds(sid * rows_per_tile, rows_per_tile)],
            out_hbm.at[pl.ds(cid * NP + sid * rows_per_tile, rows_per_tile)],
        )

    return deg_kernel


GL = NBUF * CH


def _make_agg_kernel(Epad, NP, D):
    cpt = Epad // (NSUB * CH)
    assert cpt % (2 * NBUF) == 0
    ngrp = cpt // NBUF

    ept = cpt * CH

    @functools.partial(
        pl.kernel,
        out_type=jax.ShapeDtypeStruct((2 * NP, D), jnp.float32),
        mesh=_sc_mesh(),
        scratch_types=(
            [pltpu.VMEM((GL,), jnp.int32) for _ in range(2)]
            + [pltpu.VMEM((GL,), jnp.int32) for _ in range(2)]
            + [pltpu.VMEM((CH, D), jnp.float32) for _ in range(NBUF)]
            + [pltpu.VMEM_SHARED((NP, D), jnp.float32)]
            + [pltpu.SemaphoreType.DMA for _ in range(NBUF)]
            + [pltpu.SemaphoreType.DMA for _ in range(2)]
            + [pltpu.SemaphoreType.DMA for _ in range(2)]
        ),
    )
    def agg_kernel(h_hbm, srcadj_hbm, dst_hbm, out_hbm, *refs):
        sg = refs[0:2]
        dg = refs[2:4]
        rows = refs[4:4 + NBUF]
        acc_sh = refs[4 + NBUF]
        gsem = refs[5 + NBUF:5 + 2 * NBUF]
        esem = refs[5 + 2 * NBUF:7 + 2 * NBUF]
        dsem = refs[7 + 2 * NBUF:9 + 2 * NBUF]
        cid = lax.axis_index("c")
        sid = lax.axis_index("s")
        rows_per_tile = NP // NSUB
        sbase = cid * Epad + sid * ept
        dbase = sid * ept

        def zfill(i, _):
            for j in range(D // L):
                rows[0][i, pl.ds(j * L, L)] = jnp.zeros((L,), jnp.float32)
            return 0

        lax.fori_loop(0, CH, zfill, 0)
        for k in range(0, rows_per_tile, CH):
            n = min(CH, rows_per_tile - k)
            pltpu.sync_copy(rows[0].at[pl.ds(0, n)],
                            acc_sh.at[pl.ds(sid * rows_per_tile + k, n)])
        plsc.subcore_barrier()

        def grp_copy(g, p):
            pltpu.async_copy(srcadj_hbm.at[pl.ds(sbase + g * GL, GL)], sg[p], esem[p])
            pltpu.async_copy(dst_hbm.at[pl.ds(dbase + g * GL, GL)], dg[p], dsem[p])

        def grp_wait(g, p):
            pltpu.make_async_copy(srcadj_hbm.at[pl.ds(sbase + g * GL, GL)], sg[p], esem[p]).wait()
            pltpu.make_async_copy(dst_hbm.at[pl.ds(dbase + g * GL, GL)], dg[p], dsem[p]).wait()

        def gather(b, p):
            pltpu.async_copy(h_hbm.at[sg[p].at[pl.ds(b * CH, CH)]], rows[b], gsem[b])

        def gather_wait(b, p):
            pltpu.make_async_copy(h_hbm.at[sg[p].at[pl.ds(b * CH, CH)]], rows[b], gsem[b]).wait()

        grp_copy(0, 0)
        grp_copy(1, 1)
        grp_wait(0, 0)
        for b in range(NBUF):
            gather(b, 0)

        def body(g2, _):
            for p in range(2):
                g = g2 * 2 + p
                for b in range(NBUF):
                    gather_wait(b, p)
                    pltpu.sync_copy(rows[b], acc_sh.at[dg[p].at[pl.ds(b * CH, CH)]],
                                    add=True)
                    if b == 0:
                        @pl.when(g + 1 < ngrp)
                        def _():
                            grp_wait(g + 1, 1 - p)

                    @pl.when(g + 1 < ngrp)
                    def _():
                        gather(b, 1 - p)

                @pl.when(g + 2 < ngrp)
                def _():
                    grp_copy(g + 2, p)

            return 0

        lax.fori_loop(0, ngrp // 2, body, 0)
        plsc.subcore_barrier()
        pltpu.sync_copy(
            acc_sh.at[pl.ds(sid * rows_per_tile, rows_per_tile)],
            out_hbm.at[pl.ds(cid * NP + sid * rows_per_tile, rows_per_tile)],
        )

    return agg_kernel


def _norms(cnt_blk):
    ns = lax.rsqrt(1.0 + cnt_blk[0, :, 0:1])
    nd = lax.rsqrt(1.0 + cnt_blk[1, :, 0:1])
    return ns, nd


def _layer0_body(x_ref, cnt_ref, w_ref, o_ref):
    ns, _ = _norms(cnt_ref[...])
    h = jnp.dot(x_ref[...] * ns, w_ref[...], preferred_element_type=jnp.float32)
    o_ref[0] = h[:, :128]
    o_ref[1] = h[:, 128:]


def _layer1_body(agg_ref, h_ref, cnt_ref, w_ref, b_ref, o_ref):
    ns, nd = _norms(cnt_ref[...])
    a = agg_ref[...] + h_ref[...]
    emb = jnp.concatenate([a[0], a[1]], axis=1) * nd + b_ref[...]
    emb = jnp.where(emb > 0, emb, emb * 0.01)
    h1 = jnp.dot(emb * ns, w_ref[...], preferred_element_type=jnp.float32)
    o_ref[0] = h1[:, :128]
    o_ref[1] = h1[:, 128:]


def _head_body(agg_ref, h_ref, cnt_ref, wp_ref, b1_ref, bp_ref, o_ref):
    _, nd = _norms(cnt_ref[...])
    a = agg_ref[...] + h_ref[...]
    emb = jnp.concatenate([a[0], a[1]], axis=1) * nd + b1_ref[...]
    o_ref[...] = jnp.dot(emb, wp_ref[...], preferred_element_type=jnp.float32) + bp_ref[...]


def _tc_call(body, grid, in_specs, out_specs, out_shape):
    return pl.pallas_call(
        body,
        grid=grid,
        in_specs=in_specs,
        out_specs=out_specs,
        out_shape=out_shape,
    )


def kernel(features, edge_index, W0, b0, W1, b1, Wp, bp):
    N, D_IN = features.shape
    E = edge_index.shape[1]
    D_LAT = W0.shape[1]
    D = D_LAT // 2

    NP = ((N + 1 + 2047) // 2048) * 2048
    EG = NSUB * CH * NBUF * 2
    Epad = ((E + EG - 1) // EG) * EG

    ei = edge_index.astype(jnp.int32)
    padi = jnp.full((Epad - E,), N, jnp.int32)
    src_p = jnp.concatenate([ei[0], padi])
    dst_p = jnp.concatenate([ei[1], padi])
    edge_flat = jnp.concatenate([src_p, dst_p])
    src_adj = jnp.concatenate([src_p, src_p + NP])

    x_p = jnp.pad(features, ((0, NP - N), (0, 0)))

    deg_k = _make_degree_kernel(Epad, NP)
    agg_k = _make_agg_kernel(Epad, NP, D)

    cnt = deg_k(edge_flat).reshape(2, NP, CL)

    B = 1024
    nblk = NP // B
    row3_spec = pl.BlockSpec((2, B, 128), lambda i: (0, i, 0))
    cnt_spec = pl.BlockSpec((2, B, CL), lambda i: (0, i, 0))

    h0 = _tc_call(
        _layer0_body,
        grid=(nblk,),
        in_specs=[
            pl.BlockSpec((B, D_IN), lambda i: (i, 0)),
            cnt_spec,
            pl.BlockSpec((D_IN, D_LAT), lambda i: (0, 0)),
        ],
        out_specs=row3_spec,
        out_shape=jax.ShapeDtypeStruct((2, NP, 128), jnp.float32),
    )(x_p, cnt, W0)

    agg0 = agg_k(h0.reshape(2 * NP, 128), src_adj, dst_p).reshape(2, NP, 128)

    h1 = _tc_call(
        _layer1_body,
        grid=(nblk,),
        in_specs=[
            row3_spec,
            row3_spec,
            cnt_spec,
            pl.BlockSpec((D_LAT, D_LAT), lambda i: (0, 0)),
            pl.BlockSpec((1, D_LAT), lambda i: (0, 0)),
        ],
        out_specs=row3_spec,
        out_shape=jax.ShapeDtypeStruct((2, NP, 128), jnp.float32),
    )(agg0, h0, cnt, W1, b0.reshape(1, D_LAT))

    agg1 = agg_k(h1.reshape(2 * NP, 128), src_adj, dst_p).reshape(2, NP, 128)

    logits = _tc_call(
        _head_body,
        grid=(nblk,),
        in_specs=[
            row3_spec,
            row3_spec,
            cnt_spec,
            pl.BlockSpec((D_LAT, 1), lambda i: (0, 0)),
            pl.BlockSpec((1, D_LAT), lambda i: (0, 0)),
            pl.BlockSpec((1, 1), lambda i: (0, 0)),
        ],
        out_specs=pl.BlockSpec((B, 1), lambda i: (i, 0)),
        out_shape=jax.ShapeDtypeStruct((NP, 1), jnp.float32),
    )(agg1, h1, cnt, Wp, b1.reshape(1, D_LAT), bp.reshape(1, 1))

    return logits[:N]


# speedup vs baseline: 1.9219x; 1.9219x over previous
import functools

import jax
import jax.numpy as jnp
from jax import lax
from jax.experimental import pallas as pl
from jax.experimental.pallas import tpu as pltpu
from jax.experimental.pallas import tpu_sc as plsc

L = 16
NSUB = 16
NCORE = 2
CH = 88
NBUF = 4


def _sc_mesh():
    return plsc.VectorSubcoreMesh(
        core_axis_name="c", subcore_axis_name="s",
        num_cores=NCORE, num_subcores=NSUB,
    )


CL = 128


def _make_degree_kernel(Epad, NP):
    cpt = Epad // (NSUB * CH)

    @functools.partial(
        pl.kernel,
        out_type=jax.ShapeDtypeStruct((2 * NP, CL), jnp.float32),
        mesh=_sc_mesh(),
        scratch_types=[
            pltpu.VMEM((CH,), jnp.int32),
            pltpu.VMEM((CH, CL), jnp.float32),
            pltpu.VMEM((CH, CL), jnp.float32),
            pltpu.VMEM_SHARED((NP, CL), jnp.float32),
        ],
    )
    def deg_kernel(edges_hbm, out_hbm, idx_v, ones_v, zero_v, bins_sh):
        cid = lax.axis_index("c")
        sid = lax.axis_index("s")
        rows_per_tile = NP // NSUB

        def fill(i, _):
            for j in range(CL // L):
                ones_v[i, pl.ds(j * L, L)] = jnp.full((L,), 1.0, jnp.float32)
                zero_v[i, pl.ds(j * L, L)] = jnp.zeros((L,), jnp.float32)
            return 0

        lax.fori_loop(0, CH, fill, 0)
        for k in range(0, rows_per_tile, CH):
            n = min(CH, rows_per_tile - k)
            pltpu.sync_copy(zero_v.at[pl.ds(0, n)],
                            bins_sh.at[pl.ds(sid * rows_per_tile + k, n)])
        plsc.subcore_barrier()

        ebase = cid * Epad + sid * (cpt * CH)

        def body(i, _):
            pltpu.sync_copy(edges_hbm.at[pl.ds(ebase + i * CH, CH)], idx_v)
            pltpu.sync_copy(ones_v, bins_sh.at[idx_v], add=True)
            return 0

        lax.fori_loop(0, cpt, body, 0)
        plsc.subcore_barrier()
        pltpu.sync_copy(
            bins_sh.at[pl.ds(sid * rows_per_tile, rows_per_tile)],
            out_hbm.at[pl.ds(cid * NP + sid * rows_per_tile, rows_per_tile)],
        )

    return deg_kernel


def _make_agg_kernel(Epad, NP, D):
    cpt = Epad // (NSUB * CH)
    assert cpt % NBUF == 0

    ept = cpt * CH

    @functools.partial(
        pl.kernel,
        out_type=jax.ShapeDtypeStruct((2 * NP, D), jnp.float32),
        mesh=_sc_mesh(),
        scratch_types=(
            [pltpu.VMEM((CH,), jnp.int32) for _ in range(NBUF)]
            + [pltpu.VMEM((CH,), jnp.int32) for _ in range(NBUF)]
            + [pltpu.VMEM((CH, D), jnp.float32) for _ in range(NBUF)]
            + [pltpu.VMEM_SHARED((NP, D), jnp.float32)]
            + [pltpu.SemaphoreType.DMA for _ in range(3 * NBUF)]
        ),
    )
    def agg_kernel(h_hbm, srcadj_hbm, dst_hbm, out_hbm, *refs):
        sidx = refs[0:NBUF]
        didx = refs[NBUF:2 * NBUF]
        rows = refs[2 * NBUF:3 * NBUF]
        acc_sh = refs[3 * NBUF]
        gsem = refs[3 * NBUF + 1:3 * NBUF + 1 + NBUF]
        esem = refs[3 * NBUF + 1 + NBUF:3 * NBUF + 1 + 2 * NBUF]
        dsem = refs[3 * NBUF + 1 + 2 * NBUF:3 * NBUF + 1 + 3 * NBUF]
        cid = lax.axis_index("c")
        sid = lax.axis_index("s")
        rows_per_tile = NP // NSUB
        sbase = cid * Epad + sid * ept
        dbase = sid * ept

        def zfill(i, _):
            for j in range(D // L):
                rows[0][i, pl.ds(j * L, L)] = jnp.zeros((L,), jnp.float32)
            return 0

        lax.fori_loop(0, CH, zfill, 0)
        for k in range(0, rows_per_tile, CH):
            n = min(CH, rows_per_tile - k)
            pltpu.sync_copy(rows[0].at[pl.ds(0, n)],
                            acc_sh.at[pl.ds(sid * rows_per_tile + k, n)])
        plsc.subcore_barrier()

        def sidx_copy(i, b):
            pltpu.async_copy(srcadj_hbm.at[pl.ds(sbase + i * CH, CH)], sidx[b], esem[b])

        def sidx_wait(i, b):
            pltpu.make_async_copy(srcadj_hbm.at[pl.ds(sbase + i * CH, CH)], sidx[b], esem[b]).wait()

        def didx_copy(i, b):
            pltpu.async_copy(dst_hbm.at[pl.ds(dbase + i * CH, CH)], didx[b], dsem[b])

        def didx_wait(i, b):
            pltpu.make_async_copy(dst_hbm.at[pl.ds(dbase + i * CH, CH)], didx[b], dsem[b]).wait()

        def gather(b):
            pltpu.async_copy(h_hbm.at[sidx[b]], rows[b], gsem[b])

        def gather_wait(b):
            pltpu.make_async_copy(h_hbm.at[sidx[b]], rows[b], gsem[b]).wait()

        for b in range(NBUF):
            sidx_copy(b, b)
            didx_copy(b, b)
        for b in range(NBUF):
            sidx_wait(b, b)
            gather(b)

        def body(ig, _):
            for b in range(NBUF):
                i = ig * NBUF + b
                gather_wait(b)
                nxt = i + NBUF

                @pl.when(nxt < cpt)
                def _():
                    sidx_copy(nxt, b)

                didx_wait(i, b)
                pltpu.sync_copy(rows[b], acc_sh.at[didx[b]], add=True)

                @pl.when(nxt < cpt)
                def _():
                    didx_copy(nxt, b)
                    sidx_wait(nxt, b)
                    gather(b)

            return 0

        lax.fori_loop(0, cpt // NBUF, body, 0)
        plsc.subcore_barrier()
        pltpu.sync_copy(
            acc_sh.at[pl.ds(sid * rows_per_tile, rows_per_tile)],
            out_hbm.at[pl.ds(cid * NP + sid * rows_per_tile, rows_per_tile)],
        )

    return agg_kernel


def _norms(cnt_blk):
    ns = lax.rsqrt(1.0 + cnt_blk[0, :, 0:1])
    nd = lax.rsqrt(1.0 + cnt_blk[1, :, 0:1])
    return ns, nd


def _layer0_body(x_ref, cnt_ref, w_ref, o_ref):
    ns, _ = _norms(cnt_ref[...])
    h = jnp.dot(x_ref[...] * ns, w_ref[...], preferred_element_type=jnp.float32)
    o_ref[0] = h[:, :128]
    o_ref[1] = h[:, 128:]


def _layer1_body(agg_ref, h_ref, cnt_ref, w_ref, b_ref, o_ref):
    ns, nd = _norms(cnt_ref[...])
    a = agg_ref[...] + h_ref[...]
    emb = jnp.concatenate([a[0], a[1]], axis=1) * nd + b_ref[...]
    emb = jnp.where(emb > 0, emb, emb * 0.01)
    h1 = jnp.dot(emb * ns, w_ref[...], preferred_element_type=jnp.float32)
    o_ref[0] = h1[:, :128]
    o_ref[1] = h1[:, 128:]


def _head_body(agg_ref, h_ref, cnt_ref, wp_ref, b1_ref, bp_ref, o_ref):
    _, nd = _norms(cnt_ref[...])
    a = agg_ref[...] + h_ref[...]
    emb = jnp.concatenate([a[0], a[1]], axis=1) * nd + b1_ref[...]
    o_ref[...] = jnp.dot(emb, wp_ref[...], preferred_element_type=jnp.float32) + bp_ref[...]


def _tc_call(body, grid, in_specs, out_specs, out_shape):
    return pl.pallas_call(
        body,
        grid=grid,
        in_specs=in_specs,
        out_specs=out_specs,
        out_shape=out_shape,
    )


def kernel(features, edge_index, W0, b0, W1, b1, Wp, bp):
    N, D_IN = features.shape
    E = edge_index.shape[1]
    D_LAT = W0.shape[1]
    D = D_LAT // 2

    NP = ((N + 1 + 2047) // 2048) * 2048
    EG = NSUB * CH * NBUF
    Epad = ((E + EG - 1) // EG) * EG

    ei = edge_index.astype(jnp.int32)
    padi = jnp.full((Epad - E,), N, jnp.int32)
    src_p = jnp.concatenate([ei[0], padi])
    dst_p = jnp.concatenate([ei[1], padi])
    edge_flat = jnp.concatenate([src_p, dst_p])
    src_adj = jnp.concatenate([src_p, src_p + NP])

    x_p = jnp.pad(features, ((0, NP - N), (0, 0)))

    deg_k = _make_degree_kernel(Epad, NP)
    agg_k = _make_agg_kernel(Epad, NP, D)

    cnt = deg_k(edge_flat).reshape(2, NP, CL)

    B = 1024
    nblk = NP // B
    row3_spec = pl.BlockSpec((2, B, 128), lambda i: (0, i, 0))
    cnt_spec = pl.BlockSpec((2, B, CL), lambda i: (0, i, 0))

    h0 = _tc_call(
        _layer0_body,
        grid=(nblk,),
        in_specs=[
            pl.BlockSpec((B, D_IN), lambda i: (i, 0)),
            cnt_spec,
            pl.BlockSpec((D_IN, D_LAT), lambda i: (0, 0)),
        ],
        out_specs=row3_spec,
        out_shape=jax.ShapeDtypeStruct((2, NP, 128), jnp.float32),
    )(x_p, cnt, W0)

    agg0 = agg_k(h0.reshape(2 * NP, 128), src_adj, dst_p).reshape(2, NP, 128)

    h1 = _tc_call(
        _layer1_body,
        grid=(nblk,),
        in_specs=[
            row3_spec,
            row3_spec,
            cnt_spec,
            pl.BlockSpec((D_LAT, D_LAT), lambda i: (0, 0)),
            pl.BlockSpec((1, D_LAT), lambda i: (0, 0)),
        ],
        out_specs=row3_spec,
        out_shape=jax.ShapeDtypeStruct((2, NP, 128), jnp.float32),
    )(agg0, h0, cnt, W1, b0.reshape(1, D_LAT))

    agg1 = agg_k(h1.reshape(2 * NP, 128), src_adj, dst_p).reshape(2, NP, 128)

    logits = _tc_call(
        _head_body,
        grid=(nblk,),
        in_specs=[
            row3_spec,
            row3_spec,
            cnt_spec,
            pl.BlockSpec((D_LAT, 1), lambda i: (0, 0)),
            pl.BlockSpec((1, D_LAT), lambda i: (0, 0)),
            pl.BlockSpec((1, 1), lambda i: (0, 0)),
        ],
        out_specs=pl.BlockSpec((B, 1), lambda i: (i, 0)),
        out_shape=jax.ShapeDtypeStruct((NP, 1), jnp.float32),
    )(agg1, h1, cnt, Wp, b1.reshape(1, D_LAT), bp.reshape(1, 1))

    return logits[:N]


# speedup vs baseline: 2.1915x; 1.1402x over previous
import functools

import jax
import jax.numpy as jnp
from jax import lax
from jax.experimental import pallas as pl
from jax.experimental.pallas import tpu as pltpu
from jax.experimental.pallas import tpu_sc as plsc

L = 16
NSUB = 16
NCORE = 2
CH = 88
NBUF = 4


def _sc_mesh():
    return plsc.VectorSubcoreMesh(
        core_axis_name="c", subcore_axis_name="s",
        num_cores=NCORE, num_subcores=NSUB,
    )


CL = 128


def _make_degree_kernel(Epad, NP):
    cpt = Epad // (NSUB * CH)
    assert cpt % 2 == 0

    @functools.partial(
        pl.kernel,
        out_type=jax.ShapeDtypeStruct((2 * NP, CL), jnp.float32),
        mesh=_sc_mesh(),
        scratch_types=[
            pltpu.VMEM((CH,), jnp.int32),
            pltpu.VMEM((CH,), jnp.int32),
            pltpu.VMEM((CH, CL), jnp.float32),
            pltpu.VMEM((CH, CL), jnp.float32),
            pltpu.VMEM_SHARED((NP, CL), jnp.float32),
            pltpu.SemaphoreType.DMA,
            pltpu.SemaphoreType.DMA,
        ],
    )
    def deg_kernel(edges_hbm, out_hbm, idx0, idx1, ones_v, zero_v, bins_sh,
                   is0, is1):
        cid = lax.axis_index("c")
        sid = lax.axis_index("s")
        rows_per_tile = NP // NSUB
        idx = (idx0, idx1)
        isem = (is0, is1)

        def fill(i, _):
            for j in range(CL // L):
                ones_v[i, pl.ds(j * L, L)] = jnp.full((L,), 1.0, jnp.float32)
                zero_v[i, pl.ds(j * L, L)] = jnp.zeros((L,), jnp.float32)
            return 0

        lax.fori_loop(0, CH, fill, 0)
        for k in range(0, rows_per_tile, CH):
            n = min(CH, rows_per_tile - k)
            pltpu.sync_copy(zero_v.at[pl.ds(0, n)],
                            bins_sh.at[pl.ds(sid * rows_per_tile + k, n)])
        plsc.subcore_barrier()

        ebase = cid * Epad + sid * (cpt * CH)

        def idx_copy(i, b):
            pltpu.async_copy(edges_hbm.at[pl.ds(ebase + i * CH, CH)], idx[b], isem[b])

        def idx_wait(i, b):
            pltpu.make_async_copy(edges_hbm.at[pl.ds(ebase + i * CH, CH)], idx[b], isem[b]).wait()

        idx_copy(0, 0)
        idx_copy(1, 1)

        def body(i2, _):
            for b in range(2):
                i = i2 * 2 + b
                idx_wait(i, b)
                pltpu.sync_copy(ones_v, bins_sh.at[idx[b]], add=True)

                @pl.when(i + 2 < cpt)
                def _():
                    idx_copy(i + 2, b)

            return 0

        lax.fori_loop(0, cpt // 2, body, 0)
        plsc.subcore_barrier()
        pltpu.sync_copy(
            bins_sh.at[pl.ds(sid * rows_per_tile, rows_per_tile)],
            out_hbm.at[pl.ds(cid * NP + sid * rows_per_tile, rows_per_tile)],
        )

    return deg_kernel


def _make_agg_kernel(Epad, NP, D):
    cpt = Epad // (NSUB * CH)
    assert cpt % NBUF == 0

    ept = cpt * CH

    @functools.partial(
        pl.kernel,
        out_type=jax.ShapeDtypeStruct((2 * NP, D), jnp.float32),
        mesh=_sc_mesh(),
        scratch_types=(
            [pltpu.VMEM((CH,), jnp.int32) for _ in range(NBUF)]
            + [pltpu.VMEM((CH,), jnp.int32) for _ in range(NBUF)]
            + [pltpu.VMEM((CH, D), jnp.float32) for _ in range(NBUF)]
            + [pltpu.VMEM_SHARED((NP, D), jnp.float32)]
            + [pltpu.SemaphoreType.DMA for _ in range(3 * NBUF)]
        ),
    )
    def agg_kernel(h_hbm, srcadj_hbm, dst_hbm, out_hbm, *refs):
        sidx = refs[0:NBUF]
        didx = refs[NBUF:2 * NBUF]
        rows = refs[2 * NBUF:3 * NBUF]
        acc_sh = refs[3 * NBUF]
        gsem = refs[3 * NBUF + 1:3 * NBUF + 1 + NBUF]
        esem = refs[3 * NBUF + 1 + NBUF:3 * NBUF + 1 + 2 * NBUF]
        dsem = refs[3 * NBUF + 1 + 2 * NBUF:3 * NBUF + 1 + 3 * NBUF]
        cid = lax.axis_index("c")
        sid = lax.axis_index("s")
        rows_per_tile = NP // NSUB
        sbase = cid * Epad + sid * ept
        dbase = sid * ept

        def zfill(i, _):
            for j in range(D // L):
                rows[0][i, pl.ds(j * L, L)] = jnp.zeros((L,), jnp.float32)
            return 0

        lax.fori_loop(0, CH, zfill, 0)
        for k in range(0, rows_per_tile, CH):
            n = min(CH, rows_per_tile - k)
            pltpu.sync_copy(rows[0].at[pl.ds(0, n)],
                            acc_sh.at[pl.ds(sid * rows_per_tile + k, n)])
        plsc.subcore_barrier()

        def sidx_copy(i, b):
            pltpu.async_copy(srcadj_hbm.at[pl.ds(sbase + i * CH, CH)], sidx[b], esem[b])

        def sidx_wait(i, b):
            pltpu.make_async_copy(srcadj_hbm.at[pl.ds(sbase + i * CH, CH)], sidx[b], esem[b]).wait()

        def didx_copy(i, b):
            pltpu.async_copy(dst_hbm.at[pl.ds(dbase + i * CH, CH)], didx[b], dsem[b])

        def didx_wait(i, b):
            pltpu.make_async_copy(dst_hbm.at[pl.ds(dbase + i * CH, CH)], didx[b], dsem[b]).wait()

        def gather(b):
            pltpu.async_copy(h_hbm.at[sidx[b]], rows[b], gsem[b])

        def gather_wait(b):
            pltpu.make_async_copy(h_hbm.at[sidx[b]], rows[b], gsem[b]).wait()

        for b in range(NBUF):
            sidx_copy(b, b)
            didx_copy(b, b)
        for b in range(NBUF):
            sidx_wait(b, b)
            gather(b)

        def body(ig, _):
            for b in range(NBUF):
                i = ig * NBUF + b
                gather_wait(b)
                nxt = i + NBUF

                @pl.when(nxt < cpt)
                def _():
                    sidx_copy(nxt, b)

                didx_wait(i, b)
                pltpu.sync_copy(rows[b], acc_sh.at[didx[b]], add=True)

                @pl.when(nxt < cpt)
                def _():
                    didx_copy(nxt, b)
                    sidx_wait(nxt, b)
                    gather(b)

            return 0

        lax.fori_loop(0, cpt // NBUF, body, 0)
        plsc.subcore_barrier()
        pltpu.sync_copy(
            acc_sh.at[pl.ds(sid * rows_per_tile, rows_per_tile)],
            out_hbm.at[pl.ds(cid * NP + sid * rows_per_tile, rows_per_tile)],
        )

    return agg_kernel


def _norms(cnt_blk):
    ns = lax.rsqrt(1.0 + cnt_blk[0, :, 0:1])
    nd = lax.rsqrt(1.0 + cnt_blk[1, :, 0:1])
    return ns, nd


def _layer0_body(x_ref, cnt_ref, w_ref, o_ref):
    ns, _ = _norms(cnt_ref[...])
    h = jnp.dot(x_ref[...] * ns, w_ref[...], preferred_element_type=jnp.float32)
    o_ref[0] = h[:, :128]
    o_ref[1] = h[:, 128:]


def _layer1_body(agg_ref, h_ref, cnt_ref, w_ref, b_ref, o_ref):
    ns, nd = _norms(cnt_ref[...])
    a = agg_ref[...] + h_ref[...]
    emb = jnp.concatenate([a[0], a[1]], axis=1) * nd + b_ref[...]
    emb = jnp.where(emb > 0, emb, emb * 0.01)
    h1 = jnp.dot(emb * ns, w_ref[...], preferred_element_type=jnp.float32)
    o_ref[0] = h1[:, :128]
    o_ref[1] = h1[:, 128:]


def _head_body(agg_ref, h_ref, cnt_ref, wp_ref, b1_ref, bp_ref, o_ref):
    _, nd = _norms(cnt_ref[...])
    a = agg_ref[...] + h_ref[...]
    emb = jnp.concatenate([a[0], a[1]], axis=1) * nd + b1_ref[...]
    o_ref[...] = jnp.dot(emb, wp_ref[...], preferred_element_type=jnp.float32) + bp_ref[...]


def _tc_call(body, grid, in_specs, out_specs, out_shape):
    return pl.pallas_call(
        body,
        grid=grid,
        in_specs=in_specs,
        out_specs=out_specs,
        out_shape=out_shape,
    )


def kernel(features, edge_index, W0, b0, W1, b1, Wp, bp):
    N, D_IN = features.shape
    E = edge_index.shape[1]
    D_LAT = W0.shape[1]
    D = D_LAT // 2

    NP = ((N + 1 + 2047) // 2048) * 2048
    EG = NSUB * CH * NBUF
    Epad = ((E + EG - 1) // EG) * EG

    ei = edge_index.astype(jnp.int32)
    padi = jnp.full((Epad - E,), N, jnp.int32)
    src_p = jnp.concatenate([ei[0], padi])
    dst_p = jnp.concatenate([ei[1], padi])
    edge_flat = jnp.concatenate([src_p, dst_p])
    src_adj = jnp.concatenate([src_p, src_p + NP])

    x_p = jnp.pad(features, ((0, NP - N), (0, 0)))

    deg_k = _make_degree_kernel(Epad, NP)
    agg_k = _make_agg_kernel(Epad, NP, D)

    cnt = deg_k(edge_flat).reshape(2, NP, CL)

    B = 1024
    nblk = NP // B
    row3_spec = pl.BlockSpec((2, B, 128), lambda i: (0, i, 0))
    cnt_spec = pl.BlockSpec((2, B, CL), lambda i: (0, i, 0))

    h0 = _tc_call(
        _layer0_body,
        grid=(nblk,),
        in_specs=[
            pl.BlockSpec((B, D_IN), lambda i: (i, 0)),
            cnt_spec,
            pl.BlockSpec((D_IN, D_LAT), lambda i: (0, 0)),
        ],
        out_specs=row3_spec,
        out_shape=jax.ShapeDtypeStruct((2, NP, 128), jnp.float32),
    )(x_p, cnt, W0)

    agg0 = agg_k(h0.reshape(2 * NP, 128), src_adj, dst_p).reshape(2, NP, 128)

    h1 = _tc_call(
        _layer1_body,
        grid=(nblk,),
        in_specs=[
            row3_spec,
            row3_spec,
            cnt_spec,
            pl.BlockSpec((D_LAT, D_LAT), lambda i: (0, 0)),
            pl.BlockSpec((1, D_LAT), lambda i: (0, 0)),
        ],
        out_specs=row3_spec,
        out_shape=jax.ShapeDtypeStruct((2, NP, 128), jnp.float32),
    )(agg0, h0, cnt, W1, b0.reshape(1, D_LAT))

    agg1 = agg_k(h1.reshape(2 * NP, 128), src_adj, dst_p).reshape(2, NP, 128)

    logits = _tc_call(
        _head_body,
        grid=(nblk,),
        in_specs=[
            row3_spec,
            row3_spec,
            cnt_spec,
            pl.BlockSpec((D_LAT, 1), lambda i: (0, 0)),
            pl.BlockSpec((1, D_LAT), lambda i: (0, 0)),
            pl.BlockSpec((1, 1), lambda i: (0, 0)),
        ],
        out_specs=pl.BlockSpec((B, 1), lambda i: (i, 0)),
        out_shape=jax.ShapeDtypeStruct((NP, 1), jnp.float32),
    )(agg1, h1, cnt, Wp, b1.reshape(1, D_LAT), bp.reshape(1, 1))

    return logits[:N]
